# baseline (speedup 1.0000x reference)
import functools

import jax
import jax.numpy as jnp
from jax import lax
from jax.experimental import pallas as pl
from jax.experimental.pallas import tpu as pltpu
from jax.experimental.pallas import tpu_sc as plsc

N_NODES = 10000
N_EDGES = 320000
D = 128
EPS = 1e-5
NEG_SLOPE = 0.01

NC, NS, L = 2, 16, 16
NW = NC * NS
CH = 72
EPW = 10080
EPAD = NW * EPW
NCH = EPW // CH
NSLOT = 5
NIDX = 2 * NSLOT
NPAD = 10240
RPT = NPAD // NS
ZCH = 128
NRC = RPT // ZCH

_mesh = plsc.VectorSubcoreMesh(
    core_axis_name="c", subcore_axis_name="s", num_cores=NC, num_subcores=NS)


@functools.partial(
    pl.kernel,
    out_type=jax.ShapeDtypeStruct((NW, NPAD), jnp.float32),
    mesh=_mesh,
    compiler_params=pltpu.CompilerParams(needs_layout_passes=False),
    scratch_types=[
        pltpu.VMEM((EPW,), jnp.int32),
        pltpu.VMEM((NPAD,), jnp.float32),
    ],
)
def _deg_kernel(dst_hbm, deg_hbm, dst_v, deg_v):
    wid = lax.axis_index("s") * NC + lax.axis_index("c")
    pltpu.sync_copy(dst_hbm.at[pl.ds(wid * EPW, EPW)], dst_v)

    zero = jnp.zeros((L,), jnp.float32)

    def _zero(i, c):
        deg_v[pl.ds(i * L, L)] = zero
        return c

    lax.fori_loop(0, NPAD // L, _zero, 0)

    ones = jnp.ones((L,), jnp.float32)

    def _count(i, c):
        idx = dst_v[pl.ds(i * L, L)]
        plsc.addupdate_scatter(deg_v, [idx], ones)
        return c

    lax.fori_loop(0, EPW // L, _count, 0)
    pltpu.sync_copy(deg_v, deg_hbm.at[wid])


@functools.partial(
    pl.kernel,
    out_type=jax.ShapeDtypeStruct((NC, N_NODES, D), jnp.float32),
    mesh=_mesh,
    compiler_params=pltpu.CompilerParams(needs_layout_passes=False),
    scratch_types=(
        [pltpu.VMEM((CH,), jnp.int32) for _ in range(NIDX)]
        + [pltpu.VMEM((CH,), jnp.int32) for _ in range(NIDX)]
        + [pltpu.VMEM((CH, D), jnp.float32) for _ in range(NSLOT)]
        + [pltpu.VMEM_SHARED((NPAD, D), jnp.float32)]
        + [pltpu.SemaphoreType.DMA] * (NIDX + 2 * NSLOT)
    ),
)
def _msg_kernel(src_hbm, dst_hbm, y_hbm, acc_hbm, *bufs):
    srci = bufs[0:NIDX]
    dsti = bufs[NIDX:2 * NIDX]
    rows = bufs[2 * NIDX:2 * NIDX + NSLOT]
    acc_s = bufs[2 * NIDX + NSLOT]
    o = 2 * NIDX + NSLOT + 1
    isem = bufs[o:o + NIDX]
    gsem = bufs[o + NIDX:o + NIDX + NSLOT]
    ssem = bufs[o + NIDX + NSLOT:o + NIDX + 2 * NSLOT]
    cid = lax.axis_index("c")
    sid = lax.axis_index("s")
    wid = sid * NC + cid
    ebase = wid * EPW

    def _start_idx(c, j):
        cb = ebase + c * CH
        pltpu.async_copy(src_hbm.at[pl.ds(cb, CH)], srci[j], isem[j])
        pltpu.async_copy(dst_hbm.at[pl.ds(cb, CH)], dsti[j], isem[j])

    def _wait_idx(j):
        pltpu.make_async_copy(src_hbm.at[pl.ds(ebase, CH)], srci[j],
                              isem[j]).wait()
        pltpu.make_async_copy(dst_hbm.at[pl.ds(ebase, CH)], dsti[j],
                              isem[j]).wait()

    def _start_g(j, k):
        pltpu.async_copy(y_hbm.at[srci[j]], rows[k], gsem[k])

    def _wait_g(k):
        pltpu.make_async_copy(y_hbm.at[srci[0]], rows[k], gsem[k]).wait()

    def _scat(j, k):
        pltpu.async_copy(rows[k], acc_s.at[dsti[j]], ssem[k], add=True)

    def _wait_s(j, k):
        pltpu.make_async_copy(rows[k], acc_s.at[dsti[j]], ssem[k]).wait()

    for j in range(NIDX):
        _start_idx(j, j)
    for k in range(2, NSLOT):
        _wait_idx(k)
        _start_g(k, k)

    zero = jnp.zeros((L,), jnp.float32)

    def _zrow(i, c):
        for j in range(D // L):
            rows[0][i, pl.ds(j * L, L)] = zero
            rows[1][i, pl.ds(j * L, L)] = zero
        return c

    lax.fori_loop(0, CH, _zrow, 0)
    rbase = sid * RPT
    for k in range(NRC):
        pltpu.sync_copy(rows[0], acc_s.at[pl.ds(rbase + k * ZCH, CH)])
        pltpu.sync_copy(rows[1].at[pl.ds(0, ZCH - CH)],
                        acc_s.at[pl.ds(rbase + k * ZCH + CH, ZCH - CH)])
    plsc.subcore_barrier()

    for k in range(2):
        _wait_idx(k)
        _start_g(k, k)

    def _body(c, m, do_idx, do_g):
        k = m % NSLOT
        _wait_g(k)
        _scat(m, k)
        _wait_s(m, k)
        if do_idx:
            _start_idx(c + NIDX, m)
        if do_g:
            _wait_idx((m + NSLOT) % NIDX)
            _start_g((m + NSLOT) % NIDX, k)

    def _ring(i10, carry):
        base = NIDX * i10
        for m in range(NIDX):
            _body(base + m, m, True, True)
        return carry

    lax.fori_loop(0, (NCH - NIDX) // NIDX, _ring, 0)
    for c in range(NCH - NIDX, NCH - NSLOT):
        _body(c, c % NIDX, False, True)
    for c in range(NCH - NSLOT, NCH):
        _wait_g(c % NSLOT)
        _scat(c % NIDX, c % NSLOT)
    for c in range(NCH - NSLOT, NCH):
        _wait_s(c % NIDX, c % NSLOT)

    plsc.subcore_barrier()

    @pl.when(sid < N_NODES // 1000)
    def _():
        xbase = sid * 1000
        for n0, n in ((0, 128), (128, 128), (256, 128), (384, 128),
                      (512, 128), (640, 128), (768, 128), (896, 104)):
            pltpu.sync_copy(acc_s.at[pl.ds(xbase + n0, n)],
                            acc_hbm.at[cid, pl.ds(xbase + n0, n)])


BR = 2000
NB = N_NODES // BR


def _dred_body(degp_ref, dsum_ref):
    dsum_ref[...] = jnp.sum(degp_ref[...], axis=0, keepdims=True)


def _mm_body(x_ref, w_ref, xl_ref):
    xl_ref[...] = jnp.dot(x_ref[...], w_ref[...],
                          preferred_element_type=jnp.float32)


def _lin_body(xl_ref, deg_ref, y_ref, dis_ref):
    dis = lax.rsqrt(deg_ref[...] + 1.0)
    y_ref[...] = xl_ref[...] * dis
    dis_ref[...] = dis


def _e1_body(a0_ref, a1_ref, y_ref, dis_ref, b_ref, z_ref, s1_ref, s2_ref):
    a = (a0_ref[...] + a1_ref[...]).reshape(BR, D)
    t = (a + y_ref[...]) * dis_ref[...] + b_ref[...]
    z = jnp.where(t >= 0.0, t, NEG_SLOPE * t)
    z_ref[...] = z
    s1_ref[...] = jnp.sum(z, axis=0).reshape(1, 1, D)
    s2_ref[...] = jnp.sum(z * z, axis=0).reshape(1, 1, D)


def _e2_body(z_ref, s1_ref, s2_ref, g_ref, bt_ref, o_ref):
    n = jnp.float32(N_NODES)
    mean = jnp.sum(s1_ref[...], axis=0) / n
    msq = jnp.sum(s2_ref[...], axis=0) / n
    var = msq - mean * mean
    rstd = lax.rsqrt(var + EPS)
    o_ref[...] = g_ref[...] * (z_ref[...] - mean) * rstd + bt_ref[...]


def kernel(x, edge_index, W, b, gamma, beta):
    npad_e = EPAD - N_EDGES
    pad_iota = jnp.arange(npad_e, dtype=jnp.int32)
    src = jnp.concatenate(
        [edge_index[0].astype(jnp.int32), pad_iota % N_NODES])
    dst = jnp.concatenate(
        [edge_index[1].astype(jnp.int32),
         N_NODES + pad_iota % (NPAD - N_NODES)])

    deg_p = _deg_kernel(dst)

    deg_sum = pl.pallas_call(
        _dred_body,
        grid=(1,),
        in_specs=[pl.BlockSpec((NW, NPAD), lambda i: (0, 0))],
        out_specs=pl.BlockSpec((1, NPAD), lambda i: (0, 0)),
        out_shape=jax.ShapeDtypeStruct((1, NPAD), jnp.float32),
    )(deg_p)[:, :N_NODES]

    xl = pl.pallas_call(
        _mm_body,
        grid=(NB,),
        in_specs=[
            pl.BlockSpec((BR, D), lambda i: (i, 0)),
            pl.BlockSpec((D, D), lambda i: (0, 0)),
        ],
        out_specs=pl.BlockSpec((BR, D), lambda i: (i, 0)),
        out_shape=jax.ShapeDtypeStruct((N_NODES, D), jnp.float32),
    )(x, W)

    y, dis = pl.pallas_call(
        _lin_body,
        grid=(NB,),
        in_specs=[
            pl.BlockSpec((BR, D), lambda i: (i, 0)),
            pl.BlockSpec((BR, 1), lambda i: (i, 0)),
        ],
        out_specs=[
            pl.BlockSpec((BR, D), lambda i: (i, 0)),
            pl.BlockSpec((BR, 1), lambda i: (i, 0)),
        ],
        out_shape=[
            jax.ShapeDtypeStruct((N_NODES, D), jnp.float32),
            jax.ShapeDtypeStruct((N_NODES, 1), jnp.float32),
        ],
    )(xl, deg_sum.reshape(N_NODES, 1))

    acc = _msg_kernel(src, dst, y)

    z, s1, s2 = pl.pallas_call(
        _e1_body,
        grid=(NB,),
        in_specs=[
            pl.BlockSpec((1, BR, D), lambda i: (0, i, 0)),
            pl.BlockSpec((1, BR, D), lambda i: (1, i, 0)),
            pl.BlockSpec((BR, D), lambda i: (i, 0)),
            pl.BlockSpec((BR, 1), lambda i: (i, 0)),
            pl.BlockSpec((1, D), lambda i: (0, 0)),
        ],
        out_specs=[
            pl.BlockSpec((BR, D), lambda i: (i, 0)),
            pl.BlockSpec((1, 1, D), lambda i: (i, 0, 0)),
            pl.BlockSpec((1, 1, D), lambda i: (i, 0, 0)),
        ],
        out_shape=[
            jax.ShapeDtypeStruct((N_NODES, D), jnp.float32),
            jax.ShapeDtypeStruct((NB, 1, D), jnp.float32),
            jax.ShapeDtypeStruct((NB, 1, D), jnp.float32),
        ],
    )(acc, acc, y, dis, b.reshape(1, D))

    out = pl.pallas_call(
        _e2_body,
        grid=(NB,),
        in_specs=[
            pl.BlockSpec((BR, D), lambda i: (i, 0)),
            pl.BlockSpec((NB, 1, D), lambda i: (0, 0, 0)),
            pl.BlockSpec((NB, 1, D), lambda i: (0, 0, 0)),
            pl.BlockSpec((1, D), lambda i: (0, 0)),
            pl.BlockSpec((1, D), lambda i: (0, 0)),
        ],
        out_specs=pl.BlockSpec((BR, D), lambda i: (i, 0)),
        out_shape=jax.ShapeDtypeStruct((N_NODES, D), jnp.float32),
    )(z, s1, s2, gamma.reshape(1, D), beta.reshape(1, D))

    return out


# speedup vs baseline: 1.0053x; 1.0053x over previous
import functools

import jax
import jax.numpy as jnp
from jax import lax
from jax.experimental import pallas as pl
from jax.experimental.pallas import tpu as pltpu
from jax.experimental.pallas import tpu_sc as plsc

N_NODES = 10000
N_EDGES = 320000
D = 128
EPS = 1e-5
NEG_SLOPE = 0.01

NC, NS, L = 2, 16, 16
NW = NC * NS
CH = 72
EPW = 10080
EPAD = NW * EPW
NCH = EPW // CH
NSLOT = 4
NPAD = 10240
RPT = NPAD // NS
ZCH = 128
NRC = RPT // ZCH

_mesh = plsc.VectorSubcoreMesh(
    core_axis_name="c", subcore_axis_name="s", num_cores=NC, num_subcores=NS)


@functools.partial(
    pl.kernel,
    out_type=jax.ShapeDtypeStruct((NW, NPAD), jnp.float32),
    mesh=_mesh,
    compiler_params=pltpu.CompilerParams(needs_layout_passes=False),
    scratch_types=[
        pltpu.VMEM((EPW,), jnp.int32),
        pltpu.VMEM((NPAD,), jnp.float32),
    ],
)
def _deg_kernel(dst_hbm, deg_hbm, dst_v, deg_v):
    wid = lax.axis_index("s") * NC + lax.axis_index("c")
    pltpu.sync_copy(dst_hbm.at[pl.ds(wid * EPW, EPW)], dst_v)

    zero = jnp.zeros((L,), jnp.float32)

    def _zero(i, c):
        deg_v[pl.ds(i * L, L)] = zero
        return c

    lax.fori_loop(0, NPAD // L, _zero, 0)

    ones = jnp.ones((L,), jnp.float32)

    def _count(i, c):
        idx = dst_v[pl.ds(i * L, L)]
        plsc.addupdate_scatter(deg_v, [idx], ones)
        return c

    lax.fori_loop(0, EPW // L, _count, 0)
    pltpu.sync_copy(deg_v, deg_hbm.at[wid])


@functools.partial(
    pl.kernel,
    out_type=jax.ShapeDtypeStruct((NC, N_NODES, D), jnp.float32),
    mesh=_mesh,
    compiler_params=pltpu.CompilerParams(needs_layout_passes=False),
    scratch_types=(
        [pltpu.VMEM((EPW,), jnp.int32)]
        + [pltpu.VMEM((CH,), jnp.int32) for _ in range(NSLOT)]
        + [pltpu.VMEM((CH, D), jnp.float32) for _ in range(NSLOT)]
        + [pltpu.VMEM_SHARED((NPAD, D), jnp.float32)]
        + [pltpu.SemaphoreType.DMA] * (3 * NSLOT)
    ),
)
def _msg_kernel(src_hbm, dst_hbm, y_hbm, acc_hbm, src_v, *bufs):
    dsti = bufs[0:NSLOT]
    rows = bufs[NSLOT:2 * NSLOT]
    acc_s = bufs[2 * NSLOT]
    dsem = bufs[2 * NSLOT + 1:3 * NSLOT + 1]
    gsem = bufs[3 * NSLOT + 1:4 * NSLOT + 1]
    ssem = bufs[4 * NSLOT + 1:5 * NSLOT + 1]
    cid = lax.axis_index("c")
    sid = lax.axis_index("s")
    wid = sid * NC + cid
    ebase = wid * EPW

    def _start(c, k):
        cb = c * CH
        pltpu.async_copy(dst_hbm.at[pl.ds(ebase + cb, CH)], dsti[k], dsem[k])
        pltpu.async_copy(y_hbm.at[src_v.at[pl.ds(cb, CH)]], rows[k], gsem[k])

    def _wait_g(k):
        pltpu.make_async_copy(dst_hbm.at[pl.ds(ebase, CH)], dsti[k],
                              dsem[k]).wait()
        pltpu.make_async_copy(y_hbm.at[src_v.at[pl.ds(0, CH)]], rows[k],
                              gsem[k]).wait()

    def _scat(k):
        pltpu.async_copy(rows[k], acc_s.at[dsti[k]], ssem[k], add=True)

    def _wait_s(k):
        pltpu.make_async_copy(rows[k], acc_s.at[dsti[k]], ssem[k]).wait()

    pltpu.sync_copy(src_hbm.at[pl.ds(ebase, EPW)], src_v)
    _start(0, 0)
    _start(1, 1)

    zero = jnp.zeros((L,), jnp.float32)

    def _zrow(i, c):
        for j in range(D // L):
            rows[2][i, pl.ds(j * L, L)] = zero
            rows[3][i, pl.ds(j * L, L)] = zero
        return c

    lax.fori_loop(0, CH, _zrow, 0)
    rbase = sid * RPT
    for k in range(NRC):
        pltpu.sync_copy(rows[2], acc_s.at[pl.ds(rbase + k * ZCH, CH)])
        pltpu.sync_copy(rows[3].at[pl.ds(0, ZCH - CH)],
                        acc_s.at[pl.ds(rbase + k * ZCH + CH, ZCH - CH)])
    plsc.subcore_barrier()

    _start(2, 2)
    _start(3, 3)

    def _ring(i5, c):
        base = NSLOT * i5
        for k in range(NSLOT):
            _wait_g(k)
            _scat(k)
            _wait_s(k)
            _start(base + k + NSLOT, k)
        return c

    lax.fori_loop(0, NCH // NSLOT - 1, _ring, 0)
    for i in range(NCH - NSLOT, NCH):
        k = i % NSLOT
        _wait_g(k)
        _scat(k)
    for i in range(NCH - NSLOT, NCH):
        _wait_s(i % NSLOT)

    plsc.subcore_barrier()

    @pl.when(sid < N_NODES // 1000)
    def _():
        xbase = sid * 1000
        for n0, n in ((0, 128), (128, 128), (256, 128), (384, 128),
                      (512, 128), (640, 128), (768, 128), (896, 104)):
            pltpu.sync_copy(acc_s.at[pl.ds(xbase + n0, n)],
                            acc_hbm.at[cid, pl.ds(xbase + n0, n)])


BR = 2000
NB = N_NODES // BR


def _dred_body(degp_ref, dsum_ref):
    dsum_ref[...] = jnp.sum(degp_ref[...], axis=0, keepdims=True)


def _mm_body(x_ref, w_ref, xl_ref):
    xl_ref[...] = jnp.dot(x_ref[...], w_ref[...],
                          preferred_element_type=jnp.float32)


def _lin_body(xl_ref, deg_ref, y_ref, dis_ref):
    dis = lax.rsqrt(deg_ref[...] + 1.0)
    y_ref[...] = xl_ref[...] * dis
    dis_ref[...] = dis


def _e1_body(a0_ref, a1_ref, y_ref, dis_ref, b_ref, z_ref, s1_ref, s2_ref):
    a = (a0_ref[...] + a1_ref[...]).reshape(BR, D)
    t = (a + y_ref[...]) * dis_ref[...] + b_ref[...]
    z = jnp.where(t >= 0.0, t, NEG_SLOPE * t)
    z_ref[...] = z
    s1_ref[...] = jnp.sum(z, axis=0).reshape(1, 1, D)
    s2_ref[...] = jnp.sum(z * z, axis=0).reshape(1, 1, D)


def _e2_body(z_ref, s1_ref, s2_ref, g_ref, bt_ref, o_ref):
    n = jnp.float32(N_NODES)
    mean = jnp.sum(s1_ref[...], axis=0) / n
    msq = jnp.sum(s2_ref[...], axis=0) / n
    var = msq - mean * mean
    rstd = lax.rsqrt(var + EPS)
    o_ref[...] = g_ref[...] * (z_ref[...] - mean) * rstd + bt_ref[...]


def kernel(x, edge_index, W, b, gamma, beta):
    npad_e = EPAD - N_EDGES
    pad_iota = jnp.arange(npad_e, dtype=jnp.int32)
    src = jnp.concatenate(
        [edge_index[0].astype(jnp.int32), pad_iota % N_NODES])
    dst = jnp.concatenate(
        [edge_index[1].astype(jnp.int32),
         N_NODES + pad_iota % (NPAD - N_NODES)])

    deg_p = _deg_kernel(dst)

    deg_sum = pl.pallas_call(
        _dred_body,
        grid=(1,),
        in_specs=[pl.BlockSpec((NW, NPAD), lambda i: (0, 0))],
        out_specs=pl.BlockSpec((1, NPAD), lambda i: (0, 0)),
        out_shape=jax.ShapeDtypeStruct((1, NPAD), jnp.float32),
    )(deg_p)[:, :N_NODES]

    xl = pl.pallas_call(
        _mm_body,
        grid=(NB,),
        in_specs=[
            pl.BlockSpec((BR, D), lambda i: (i, 0)),
            pl.BlockSpec((D, D), lambda i: (0, 0)),
        ],
        out_specs=pl.BlockSpec((BR, D), lambda i: (i, 0)),
        out_shape=jax.ShapeDtypeStruct((N_NODES, D), jnp.float32),
    )(x, W)

    y, dis = pl.pallas_call(
        _lin_body,
        grid=(NB,),
        in_specs=[
            pl.BlockSpec((BR, D), lambda i: (i, 0)),
            pl.BlockSpec((BR, 1), lambda i: (i, 0)),
        ],
        out_specs=[
            pl.BlockSpec((BR, D), lambda i: (i, 0)),
            pl.BlockSpec((BR, 1), lambda i: (i, 0)),
        ],
        out_shape=[
            jax.ShapeDtypeStruct((N_NODES, D), jnp.float32),
            jax.ShapeDtypeStruct((N_NODES, 1), jnp.float32),
        ],
    )(xl, deg_sum.reshape(N_NODES, 1))

    acc = _msg_kernel(src, dst, y)

    z, s1, s2 = pl.pallas_call(
        _e1_body,
        grid=(NB,),
        in_specs=[
            pl.BlockSpec((1, BR, D), lambda i: (0, i, 0)),
            pl.BlockSpec((1, BR, D), lambda i: (1, i, 0)),
            pl.BlockSpec((BR, D), lambda i: (i, 0)),
            pl.BlockSpec((BR, 1), lambda i: (i, 0)),
            pl.BlockSpec((1, D), lambda i: (0, 0)),
        ],
        out_specs=[
            pl.BlockSpec((BR, D), lambda i: (i, 0)),
            pl.BlockSpec((1, 1, D), lambda i: (i, 0, 0)),
            pl.BlockSpec((1, 1, D), lambda i: (i, 0, 0)),
        ],
        out_shape=[
            jax.ShapeDtypeStruct((N_NODES, D), jnp.float32),
            jax.ShapeDtypeStruct((NB, 1, D), jnp.float32),
            jax.ShapeDtypeStruct((NB, 1, D), jnp.float32),
        ],
    )(acc, acc, y, dis, b.reshape(1, D))

    out = pl.pallas_call(
        _e2_body,
        grid=(NB,),
        in_specs=[
            pl.BlockSpec((BR, D), lambda i: (i, 0)),
            pl.BlockSpec((NB, 1, D), lambda i: (0, 0, 0)),
            pl.BlockSpec((NB, 1, D), lambda i: (0, 0, 0)),
            pl.BlockSpec((1, D), lambda i: (0, 0)),
            pl.BlockSpec((1, D), lambda i: (0, 0)),
        ],
        out_specs=pl.BlockSpec((BR, D), lambda i: (i, 0)),
        out_shape=jax.ShapeDtypeStruct((N_NODES, D), jnp.float32),
    )(z, s1, s2, gamma.reshape(1, D), beta.reshape(1, D))

    return out
